# Initial kernel scaffold; baseline (speedup 1.0000x reference)
import jax
import jax.numpy as jnp
from jax.experimental import pallas as pl


def kernel(x, edge_index, Ws, bs):
    raise NotImplementedError("write your pallas kernel here")


# speedup vs baseline: 6.2081x; 6.2081x over previous
import functools

import jax
import jax.numpy as jnp
from jax import lax
from jax.experimental import pallas as pl
from jax.experimental.pallas import tpu as pltpu
from jax.experimental.pallas import tpu_sc as plsc

N_NODES = 10000
D = 128
N_EDGES = 320000

NC = 2
NS = 16
NW = NC * NS
EPW = N_EDGES // NW
CH = 80
NCH = EPW // CH
NPAD = 10240
RPT = NPAD // NS
ZR = 32

_sc_mesh = plsc.VectorSubcoreMesh(core_axis_name="c", subcore_axis_name="s")


@functools.partial(
    pl.kernel,
    mesh=_sc_mesh,
    out_type=jax.ShapeDtypeStruct((NC, NPAD, D), jnp.float32),
    scratch_types=[
        pltpu.VMEM((NCH, CH), jnp.int32),
        pltpu.VMEM((NCH, CH), jnp.int32),
        pltpu.VMEM((CH, D), jnp.float32),
        pltpu.VMEM((ZR, D), jnp.float32),
        pltpu.VMEM_SHARED((NPAD, D), jnp.float32),
        pltpu.SemaphoreType.DMA,
    ],
)
def _sc_spmm(h_hbm, src_hbm, dst_hbm, out_hbm, src_v, dst_v, rows_v, zbuf,
             acc, sem):
    cid = lax.axis_index("c")
    sid = lax.axis_index("s")
    gwid = sid * NC + cid

    for r in range(ZR):
        for l in range(D // 16):
            zbuf[r, pl.ds(l * 16, 16)] = jnp.zeros((16,), jnp.float32)
    row0 = sid * RPT
    for j in range(RPT // ZR):
        pltpu.sync_copy(zbuf, acc.at[pl.ds(row0 + j * ZR, ZR)])
    plsc.subcore_barrier()

    pltpu.sync_copy(src_hbm.at[gwid], src_v)
    pltpu.sync_copy(dst_hbm.at[gwid], dst_v)

    def body(i, carry):
        pltpu.async_copy(h_hbm.at[src_v.at[i]], rows_v, sem).wait()
        pltpu.sync_copy(rows_v, acc.at[dst_v.at[i]], add=True)
        return carry

    lax.fori_loop(0, NCH, body, 0)
    plsc.subcore_barrier()

    pltpu.sync_copy(acc.at[pl.ds(row0, RPT)],
                    out_hbm.at[cid, pl.ds(row0, RPT)])


BN = 1000


def _tc1_body(pa_ref, pb_ref, h_ref, w0_ref, w1_ref, h1_ref, acc_ref):
    h1 = pa_ref[0] + pb_ref[0]
    h1_ref[...] = h1
    acc_ref[...] = (
        jnp.dot(h_ref[...], w0_ref[...], precision=lax.Precision.HIGHEST,
                preferred_element_type=jnp.float32)
        + jnp.dot(h1, w1_ref[...], precision=lax.Precision.HIGHEST,
                  preferred_element_type=jnp.float32))


def _tc2_body(pa_ref, pb_ref, acc_ref, w2_ref, b_ref, out_ref, *, relu):
    h2 = pa_ref[0] + pb_ref[0]
    o = acc_ref[...] + jnp.dot(h2, w2_ref[...],
                               precision=lax.Precision.HIGHEST,
                               preferred_element_type=jnp.float32)
    o = o + b_ref[...]
    out_ref[...] = jnp.maximum(o, 0.0) if relu else o


_G = N_NODES // BN
_p_spec_a = pl.BlockSpec((1, BN, D), lambda i: (0, i, 0))
_p_spec_b = pl.BlockSpec((1, BN, D), lambda i: (1, i, 0))
_row_spec = pl.BlockSpec((BN, D), lambda i: (i, 0))
_w_spec = pl.BlockSpec((D, D), lambda i: (0, 0))
_b_spec = pl.BlockSpec((1, D), lambda i: (0, 0))


def _tc1(p, h, w0, w1):
    return pl.pallas_call(
        _tc1_body,
        grid=(_G,),
        in_specs=[_p_spec_a, _p_spec_b, _row_spec, _w_spec, _w_spec],
        out_specs=[_row_spec, _row_spec],
        out_shape=[jax.ShapeDtypeStruct((N_NODES, D), jnp.float32),
                   jax.ShapeDtypeStruct((N_NODES, D), jnp.float32)],
    )(p, p, h, w0, w1)


def _tc2(p, acc, w2, b, relu):
    return pl.pallas_call(
        functools.partial(_tc2_body, relu=relu),
        grid=(_G,),
        in_specs=[_p_spec_a, _p_spec_b, _row_spec, _w_spec, _b_spec],
        out_specs=_row_spec,
        out_shape=jax.ShapeDtypeStruct((N_NODES, D), jnp.float32),
    )(p, p, acc, w2, b.reshape(1, D))


def kernel(x, edge_index, Ws, bs):
    ei = edge_index.astype(jnp.int32)
    src3 = ei[0].reshape(NW, NCH, CH)
    dst3 = ei[1].reshape(NW, NCH, CH)
    h = x
    num_layers = Ws.shape[0]
    for layer in range(num_layers):
        p = _sc_spmm(h, src3, dst3)
        h1, acc = _tc1(p, h, Ws[layer, 0], Ws[layer, 1])
        p2 = _sc_spmm(h1, src3, dst3)
        h = _tc2(p2, acc, Ws[layer, 2], bs[layer], layer < num_layers - 1)
    return h
